# baseline (speedup 1.0000x reference)
import functools

import jax
import jax.numpy as jnp
from jax import lax
from jax.experimental import pallas as pl
from jax.experimental.pallas import tpu as pltpu
from jax.experimental.pallas import tpu_sc as plsc

N_NODES = 100000
N_SEG = 256
D = 256
NC = 2
NS = 16
NW = NC * NS
BLK = 32
LANES = 16
NBUF = 4
NB = -(-N_NODES // BLK)
NBROWS = NB + 1
BLOCKS_PER_W = -(-NB // NW)
BLOCKS_PER_W_RING = -(-BLOCKS_PER_W // NBUF) * NBUF
LAST_START = N_NODES - BLK
ACC_ROWS = N_SEG + 8


def _sc_segment_partials(x, batch2d):
    mesh = plsc.VectorSubcoreMesh(core_axis_name="c", subcore_axis_name="s",
                                  num_cores=NC, num_subcores=NS)

    @functools.partial(
        pl.kernel,
        out_type=jax.ShapeDtypeStruct((NW, N_SEG, D), jnp.float32),
        mesh=mesh,
        scratch_types=(
            [pltpu.VMEM((BLK * LANES,), jnp.int32)] * NBUF
            + [pltpu.VMEM((BLK, D), jnp.float32)] * NBUF
            + [pltpu.VMEM((ACC_ROWS, D), jnp.float32)]
            + [pltpu.SemaphoreType.DMA] * (2 * NBUF)
        ),
        compiler_params=pltpu.CompilerParams(needs_layout_passes=False),
    )
    def body(x_hbm, b_hbm, out_hbm, *scratch):
        idxs = scratch[:NBUF]
        bufs = scratch[NBUF:2 * NBUF]
        acc_v = scratch[2 * NBUF]
        sis = scratch[2 * NBUF + 1:2 * NBUF + 1 + NBUF]
        sxs = scratch[2 * NBUF + 1 + NBUF:]
        c = lax.axis_index("c")
        s = lax.axis_index("s")
        w = s * NC + c

        zero16 = jnp.zeros((LANES,), jnp.float32)

        def zrow(r, carry):
            for v in range(D // LANES):
                acc_v[r, pl.ds(v * LANES, LANES)] = zero16
            return carry

        lax.fori_loop(0, ACC_ROWS, zrow, 0)

        iota = lax.iota(jnp.int32, LANES)
        cols = [iota + v * LANES for v in range(D // LANES)]

        def issue(m, idxp, bufp, semi, semx):
            j = jnp.minimum(w + NW * m, NBROWS - 1)
            start = jnp.minimum(j * BLK, LAST_START)
            pltpu.async_copy(b_hbm.at[j], idxp, semi)
            pltpu.async_copy(x_hbm.at[pl.ds(start, BLK)], bufp, semx)

        def wait(idxp, bufp, semi, semx):
            pltpu.make_async_copy(b_hbm.at[0], idxp, semi).wait()
            pltpu.make_async_copy(x_hbm.at[pl.ds(0, BLK)], bufp, semx).wait()

        def process(idxp, bufp, carry):
            prev_v, accs = carry
            for r in range(BLK):
                seg_v = idxp[pl.ds(r * LANES, LANES)]
                m_new = seg_v != prev_v
                new_accs = []
                for v in range(D // LANES):
                    plsc.addupdate_scatter(acc_v, [prev_v, cols[v]],
                                           accs[v], mask=m_new)
                    row = bufp[r, pl.ds(v * LANES, LANES)]
                    new_accs.append(jnp.where(m_new, row, accs[v] + row))
                accs = tuple(new_accs)
                prev_v = seg_v
            return prev_v, accs

        for p in range(NBUF - 1):
            issue(p, idxs[p], bufs[p], sis[p], sxs[p])
        carry0 = (jnp.full((LANES,), N_SEG, jnp.int32),
                  tuple(zero16 for _ in range(D // LANES)))

        def round_(t, carry):
            base = NBUF * t
            for p in range(NBUF):
                q = (p + NBUF - 1) % NBUF
                issue(base + p + NBUF - 1, idxs[q], bufs[q], sis[q], sxs[q])
                wait(idxs[p], bufs[p], sis[p], sxs[p])
                carry = process(idxs[p], bufs[p], carry)
            return carry

        prev_v, accs = lax.fori_loop(0, BLOCKS_PER_W_RING // NBUF, round_,
                                     carry0)
        for v in range(D // LANES):
            plsc.addupdate_scatter(acc_v, [prev_v, cols[v]], accs[v])
        for p in range(NBUF - 1):
            wait(idxs[p], bufs[p], sis[p], sxs[p])

        pltpu.sync_copy(acc_v.at[pl.ds(0, N_SEG)], out_hbm.at[w])

    return body(x, batch2d)


def _tc_finish(partials, glob, w1, w2, b2d):
    def body(p_ref, g_ref, w1_ref, w2_ref, b_ref, o_ref):
        summed = jnp.sum(p_ref[...], axis=0)
        o_ref[...] = (
            lax.dot_general(g_ref[...], w1_ref[...], (((1,), (1,)), ((), ())),
                            preferred_element_type=jnp.float32)
            + lax.dot_general(summed, w2_ref[...], (((1,), (1,)), ((), ())),
                              preferred_element_type=jnp.float32)
            + b_ref[...]
        )

    return pl.pallas_call(
        body,
        out_shape=jax.ShapeDtypeStruct((N_SEG, D), jnp.float32),
    )(partials, glob, w1, w2, b2d)


def kernel(x, glob, batch, W, b):
    batch = batch.astype(jnp.int32)
    n_dup = (NB - 1) * BLK - LAST_START
    last_row = jnp.concatenate(
        [jnp.full((n_dup,), N_SEG, jnp.int32), batch[LAST_START + n_dup:]])
    dump_row = jnp.full((1, BLK), N_SEG, jnp.int32)
    b2 = jnp.concatenate(
        [batch[:(NB - 1) * BLK].reshape(NB - 1, BLK), last_row[None, :],
         dump_row],
        axis=0)
    b2e = jnp.broadcast_to(b2[:, :, None],
                           (NBROWS, BLK, LANES)).reshape(NBROWS, BLK * LANES)
    partials = _sc_segment_partials(x, b2e)
    return _tc_finish(partials, glob, W[:, :D], W[:, D:], b.reshape(1, D))


# speedup vs baseline: 1.1548x; 1.1548x over previous
import functools

import jax
import jax.numpy as jnp
from jax import lax
from jax.experimental import pallas as pl
from jax.experimental.pallas import tpu as pltpu
from jax.experimental.pallas import tpu_sc as plsc

N_NODES = 100000
N_SEG = 256
D = 256
NC = 2
NS = 16
NW = NC * NS
BLK = 32
LANES = 16
NBUF = 2
NB = -(-N_NODES // BLK)
NBROWS = NB + 1
BLOCKS_PER_W = -(-NB // NW)
BLOCKS_PER_W_RING = -(-BLOCKS_PER_W // NBUF) * NBUF
LAST_START = N_NODES - BLK
ACC_ROWS = N_SEG + 8


def _sc_segment_partials(x, batch2d):
    mesh = plsc.VectorSubcoreMesh(core_axis_name="c", subcore_axis_name="s",
                                  num_cores=NC, num_subcores=NS)

    @functools.partial(
        pl.kernel,
        out_type=jax.ShapeDtypeStruct((NW, N_SEG, D), jnp.float32),
        mesh=mesh,
        scratch_types=(
            [pltpu.VMEM((BLK * LANES,), jnp.int32)] * NBUF
            + [pltpu.VMEM((BLK, D), jnp.float32)] * NBUF
            + [pltpu.VMEM((ACC_ROWS, D), jnp.float32)]
            + [pltpu.SemaphoreType.DMA] * (2 * NBUF)
        ),
        compiler_params=pltpu.CompilerParams(needs_layout_passes=False),
    )
    def body(x_hbm, b_hbm, out_hbm, *scratch):
        idxs = scratch[:NBUF]
        bufs = scratch[NBUF:2 * NBUF]
        acc_v = scratch[2 * NBUF]
        sis = scratch[2 * NBUF + 1:2 * NBUF + 1 + NBUF]
        sxs = scratch[2 * NBUF + 1 + NBUF:]
        c = lax.axis_index("c")
        s = lax.axis_index("s")
        w = s * NC + c

        zero16 = jnp.zeros((LANES,), jnp.float32)

        def zrow(r, carry):
            for v in range(D // LANES):
                acc_v[r, pl.ds(v * LANES, LANES)] = zero16
            return carry

        lax.fori_loop(0, ACC_ROWS, zrow, 0)

        iota = lax.iota(jnp.int32, LANES)
        cols = [iota + v * LANES for v in range(D // LANES)]

        def issue(m, idxp, bufp, semi, semx):
            j = jnp.minimum(w + NW * m, NBROWS - 1)
            start = jnp.minimum(j * BLK, LAST_START)
            pltpu.async_copy(b_hbm.at[j], idxp, semi)
            pltpu.async_copy(x_hbm.at[pl.ds(start, BLK)], bufp, semx)

        def wait(idxp, bufp, semi, semx):
            pltpu.make_async_copy(b_hbm.at[0], idxp, semi).wait()
            pltpu.make_async_copy(x_hbm.at[pl.ds(0, BLK)], bufp, semx).wait()

        def process(idxp, bufp, carry):
            prev_v, accs = carry
            for r in range(BLK):
                seg_v = idxp[pl.ds(r * LANES, LANES)]
                m_new = seg_v != prev_v
                new_accs = []
                for v in range(D // LANES):
                    plsc.addupdate_scatter(acc_v, [prev_v, cols[v]],
                                           accs[v], mask=m_new)
                    row = bufp[r, pl.ds(v * LANES, LANES)]
                    new_accs.append(jnp.where(m_new, row, accs[v] + row))
                accs = tuple(new_accs)
                prev_v = seg_v
            return prev_v, accs

        for p in range(NBUF - 1):
            issue(p, idxs[p], bufs[p], sis[p], sxs[p])
        carry0 = (jnp.full((LANES,), N_SEG, jnp.int32),
                  tuple(zero16 for _ in range(D // LANES)))

        def round_(t, carry):
            base = NBUF * t
            for p in range(NBUF):
                q = (p + NBUF - 1) % NBUF
                issue(base + p + NBUF - 1, idxs[q], bufs[q], sis[q], sxs[q])
                wait(idxs[p], bufs[p], sis[p], sxs[p])
                carry = process(idxs[p], bufs[p], carry)
            return carry

        prev_v, accs = lax.fori_loop(0, BLOCKS_PER_W_RING // NBUF, round_,
                                     carry0)
        for v in range(D // LANES):
            plsc.addupdate_scatter(acc_v, [prev_v, cols[v]], accs[v])
        for p in range(NBUF - 1):
            wait(idxs[p], bufs[p], sis[p], sxs[p])

        pltpu.sync_copy(acc_v.at[pl.ds(0, N_SEG)], out_hbm.at[w])

    return body(x, batch2d)


def _tc_finish(partials, glob, w1, w2, b2d):
    def body(p_ref, g_ref, w1_ref, w2_ref, b_ref, o_ref):
        summed = jnp.sum(p_ref[...], axis=0)
        o_ref[...] = (
            lax.dot_general(g_ref[...], w1_ref[...], (((1,), (1,)), ((), ())),
                            preferred_element_type=jnp.float32)
            + lax.dot_general(summed, w2_ref[...], (((1,), (1,)), ((), ())),
                              preferred_element_type=jnp.float32)
            + b_ref[...]
        )

    return pl.pallas_call(
        body,
        out_shape=jax.ShapeDtypeStruct((N_SEG, D), jnp.float32),
    )(partials, glob, w1, w2, b2d)


def kernel(x, glob, batch, W, b):
    batch = batch.astype(jnp.int32)
    n_dup = (NB - 1) * BLK - LAST_START
    last_row = jnp.concatenate(
        [jnp.full((n_dup,), N_SEG, jnp.int32), batch[LAST_START + n_dup:]])
    dump_row = jnp.full((1, BLK), N_SEG, jnp.int32)
    b2 = jnp.concatenate(
        [batch[:(NB - 1) * BLK].reshape(NB - 1, BLK), last_row[None, :],
         dump_row],
        axis=0)
    b2e = jnp.broadcast_to(b2[:, :, None],
                           (NBROWS, BLK, LANES)).reshape(NBROWS, BLK * LANES)
    partials = _sc_segment_partials(x, b2e)
    return _tc_finish(partials, glob, W[:, :D], W[:, D:], b.reshape(1, D))
